# Initial kernel scaffold; baseline (speedup 1.0000x reference)
import functools

import jax
import jax.numpy as jnp
from jax import lax
from jax.experimental import pallas as pl
from jax.experimental.pallas import tpu as pltpu
from jax.experimental.pallas import tpu_sc as plsc

N = 10000
E = 320000
D = 128
NC = 2
NS = 16
NW = NC * NS
EPW = E // NW
CH = 100
NCH = EPW // CH
RPS = N // NS

_mesh = plsc.VectorSubcoreMesh(core_axis_name="c", subcore_axis_name="s")


@functools.partial(
    pl.kernel,
    out_type=jax.ShapeDtypeStruct((NC, 2, N, 16), jnp.float32),
    mesh=_mesh,
    scratch_types=[
        pltpu.VMEM((NCH, CH), jnp.int32),
        pltpu.VMEM((NCH, CH), jnp.int32),
        pltpu.VMEM((CH, 16), jnp.float32),
        pltpu.VMEM_SHARED((N, 16), jnp.float32),
        pltpu.VMEM_SHARED((N, 16), jnp.float32),
    ],
)
def _deg_kernel(src_hbm, dst_hbm, z16_hbm, ones_hbm, out_hbm,
                src_v, dst_v, ones_v, acc_s, acc_d):
    c = lax.axis_index("c")
    s = lax.axis_index("s")
    w = s * NC + c
    pltpu.sync_copy(src_hbm.at[w], src_v)
    pltpu.sync_copy(dst_hbm.at[w], dst_v)
    pltpu.sync_copy(ones_hbm, ones_v)
    pltpu.sync_copy(z16_hbm.at[pl.ds(s * RPS, RPS)], acc_s.at[pl.ds(s * RPS, RPS)])
    pltpu.sync_copy(z16_hbm.at[pl.ds(s * RPS, RPS)], acc_d.at[pl.ds(s * RPS, RPS)])
    plsc.subcore_barrier()

    def chunk(j, carry):
        pltpu.sync_copy(ones_v, acc_s.at[src_v.at[j]], add=True)
        pltpu.sync_copy(ones_v, acc_d.at[dst_v.at[j]], add=True)
        return carry

    lax.fori_loop(0, NCH, chunk, 0)
    plsc.subcore_barrier()
    pltpu.sync_copy(acc_s.at[pl.ds(s * RPS, RPS)], out_hbm.at[c, 0, pl.ds(s * RPS, RPS)])
    pltpu.sync_copy(acc_d.at[pl.ds(s * RPS, RPS)], out_hbm.at[c, 1, pl.ds(s * RPS, RPS)])


@functools.partial(
    pl.kernel,
    out_type=jax.ShapeDtypeStruct((NC, N, D), jnp.float32),
    mesh=_mesh,
    scratch_types=[
        pltpu.VMEM((NCH, CH), jnp.int32),
        pltpu.VMEM((NCH, CH), jnp.int32),
        pltpu.VMEM((CH, D), jnp.float32),
        pltpu.SemaphoreType.DMA,
        pltpu.VMEM_SHARED((N, D), jnp.float32),
    ],
)
def _gs_kernel(h_hbm, src_hbm, dst_hbm, z_hbm, out_hbm,
               src_v, dst_v, rows_v, sem, acc):
    c = lax.axis_index("c")
    s = lax.axis_index("s")
    w = s * NC + c
    pltpu.sync_copy(src_hbm.at[w], src_v)
    pltpu.sync_copy(dst_hbm.at[w], dst_v)
    pltpu.sync_copy(z_hbm.at[pl.ds(s * RPS, RPS)], acc.at[pl.ds(s * RPS, RPS)])
    plsc.subcore_barrier()

    def chunk(j, carry):
        pltpu.async_copy(h_hbm.at[src_v.at[j]], rows_v, sem).wait()
        pltpu.sync_copy(rows_v, acc.at[dst_v.at[j]], add=True)
        return carry

    lax.fori_loop(0, NCH, chunk, 0)
    plsc.subcore_barrier()
    pltpu.sync_copy(acc.at[pl.ds(s * RPS, RPS)], out_hbm.at[c, pl.ds(s * RPS, RPS)])


_TR = 1000


def _prep_body(x_ref, d_ref, nin_ref, nout_ref, h_ref):
    dd = d_ref[...]
    deg_out = dd[0, 0, :, 0:1] + dd[1, 0, :, 0:1]
    deg_in = dd[0, 1, :, 0:1] + dd[1, 1, :, 0:1]
    no = jnp.where(deg_out > 0, lax.rsqrt(jnp.maximum(deg_out, 1e-12)), 0.0)
    ni = jnp.where(deg_in > 0, lax.rsqrt(jnp.maximum(deg_in, 1e-12)), 0.0)
    nout_ref[...] = no
    nin_ref[...] = ni
    h_ref[...] = x_ref[...] * no


def _prep_call(x, degp):
    return pl.pallas_call(
        _prep_body,
        grid=(N // _TR,),
        in_specs=[
            pl.BlockSpec((_TR, D), lambda i: (i, 0)),
            pl.BlockSpec((NC, 2, _TR, 16), lambda i: (0, 0, i, 0)),
        ],
        out_specs=[
            pl.BlockSpec((_TR, 1), lambda i: (i, 0)),
            pl.BlockSpec((_TR, 1), lambda i: (i, 0)),
            pl.BlockSpec((_TR, D), lambda i: (i, 0)),
        ],
        out_shape=[
            jax.ShapeDtypeStruct((N, 1), jnp.float32),
            jax.ShapeDtypeStruct((N, 1), jnp.float32),
            jax.ShapeDtypeStruct((N, D), jnp.float32),
        ],
    )(x, degp)


def _layer_body(relu, scale_out, p_ref, nin_ref, nout_ref, w_ref, b_ref, o_ref):
    x = (p_ref[0] + p_ref[1]) * nin_ref[...]
    y = jnp.dot(x, w_ref[...], preferred_element_type=jnp.float32) + b_ref[...]
    if relu:
        y = jnp.maximum(y, 0.0)
    if scale_out:
        y = y * nout_ref[...]
    o_ref[...] = y


def _layer_call(p, nin, nout, W, b, relu, scale_out):
    return pl.pallas_call(
        functools.partial(_layer_body, relu, scale_out),
        grid=(N // _TR,),
        in_specs=[
            pl.BlockSpec((NC, _TR, D), lambda i: (0, i, 0)),
            pl.BlockSpec((_TR, 1), lambda i: (i, 0)),
            pl.BlockSpec((_TR, 1), lambda i: (i, 0)),
            pl.BlockSpec((D, D), lambda i: (0, 0)),
            pl.BlockSpec((1, D), lambda i: (0, 0)),
        ],
        out_specs=pl.BlockSpec((_TR, D), lambda i: (i, 0)),
        out_shape=jax.ShapeDtypeStruct((N, D), jnp.float32),
    )(p, nin, nout, W, b)


def kernel(x, edge_index, W1, b1, W2, b2, W3, b3):
    src = edge_index[0].astype(jnp.int32).reshape(NW, NCH, CH)
    dst = edge_index[1].astype(jnp.int32).reshape(NW, NCH, CH)
    z128 = jnp.zeros((N, D), jnp.float32)
    z16 = jnp.zeros((N, 16), jnp.float32)
    ones16 = jnp.ones((CH, 16), jnp.float32)
    b1r = b1.reshape(1, D)
    b2r = b2.reshape(1, D)
    b3r = b3.reshape(1, D)

    degp = _deg_kernel(src, dst, z16, ones16)
    nin, nout, h = _prep_call(x, degp)

    p = _gs_kernel(h, src, dst, z128)
    h = _layer_call(p, nin, nout, W1, b1r, relu=True, scale_out=True)
    p = _gs_kernel(h, src, dst, z128)
    h = _layer_call(p, nin, nout, W2, b2r, relu=True, scale_out=True)
    p = _gs_kernel(h, src, dst, z128)
    out = _layer_call(p, nin, nout, W3, b3r, relu=False, scale_out=False)
    return out


# speedup vs baseline: 8.4054x; 8.4054x over previous
import functools

import jax
import jax.numpy as jnp
from jax import lax
from jax.experimental import pallas as pl
from jax.experimental.pallas import tpu as pltpu
from jax.experimental.pallas import tpu_sc as plsc

N = 10000
E = 320000
D = 128
NC = 2
NS = 16
NW = NC * NS
CH = 64
NCH = 158
EP = NW * NCH * CH
CHD = 128
NCHD = EP // (NS * CHD)
NP = 10240
RPS = NP // NS

_mesh = plsc.VectorSubcoreMesh(core_axis_name="c", subcore_axis_name="s")


@functools.partial(
    pl.kernel,
    out_type=jax.ShapeDtypeStruct((NC, NP, D), jnp.float32),
    mesh=_mesh,
    scratch_types=[
        pltpu.VMEM((NCHD, CHD), jnp.int32),
        pltpu.VMEM((CHD, D), jnp.float32),
        pltpu.VMEM_SHARED((NP, D), jnp.float32),
    ],
)
def _deg_kernel(src_hbm, dst_hbm, z_hbm, ones_hbm, out_hbm,
                idx_v, buf_v, acc):
    c = lax.axis_index("c")
    s = lax.axis_index("s")

    @pl.when(c == 0)
    def _():
        pltpu.sync_copy(src_hbm.at[s], idx_v)

    @pl.when(c != 0)
    def _():
        pltpu.sync_copy(dst_hbm.at[s], idx_v)

    pltpu.sync_copy(z_hbm.at[pl.ds(0, CHD)], buf_v)
    for t in range(RPS // CHD):
        pltpu.sync_copy(buf_v, acc.at[pl.ds(s * RPS + t * CHD, CHD)])
    plsc.subcore_barrier()
    pltpu.sync_copy(ones_hbm, buf_v)

    def chunk(j, carry):
        pltpu.sync_copy(buf_v, acc.at[idx_v.at[j]], add=True)
        return carry

    lax.fori_loop(0, NCHD, chunk, 0)
    plsc.subcore_barrier()
    for t in range(RPS // CHD):
        pltpu.sync_copy(acc.at[pl.ds(s * RPS + t * CHD, CHD)], buf_v)
        pltpu.sync_copy(buf_v, out_hbm.at[c, pl.ds(s * RPS + t * CHD, CHD)])


@functools.partial(
    pl.kernel,
    out_type=jax.ShapeDtypeStruct((NC, NP, D), jnp.float32),
    mesh=_mesh,
    scratch_types=[
        pltpu.VMEM((NCH, CH), jnp.int32),
        pltpu.VMEM((CH,), jnp.int32),
        pltpu.VMEM((CH,), jnp.int32),
        pltpu.VMEM((CH,), jnp.int32),
        pltpu.VMEM((CH,), jnp.int32),
        pltpu.VMEM((CH, D), jnp.float32),
        pltpu.VMEM((CH, D), jnp.float32),
        pltpu.SemaphoreType.DMA,
        pltpu.SemaphoreType.DMA,
        pltpu.VMEM_SHARED((NP, D), jnp.float32),
    ],
)
def _gs_kernel(h_hbm, pk_hbm, z_hbm, out_hbm,
               pk_v, sa_a, da_a, sa_b, da_b, rows_a, rows_b,
               sem_a, sem_b, acc):
    c = lax.axis_index("c")
    s = lax.axis_index("s")
    w = s * NC + c
    pltpu.sync_copy(pk_hbm.at[w], pk_v)
    pltpu.sync_copy(z_hbm.at[pl.ds(0, CH)], rows_a)
    for t in range(RPS // CH):
        pltpu.sync_copy(rows_a, acc.at[pl.ds(s * RPS + t * CH, CH)])
    plsc.subcore_barrier()

    def unpack(j, sa, da):
        for q in range(CH // 16):
            v = pk_v[j, pl.ds(q * 16, 16)]
            sa[pl.ds(q * 16, 16)] = lax.bitwise_and(v, 0x3FFF)
            da[pl.ds(q * 16, 16)] = lax.shift_right_logical(v, 14)

    unpack(0, sa_a, da_a)
    pltpu.async_copy(h_hbm.at[sa_a], rows_a, sem_a)

    def body(k2, carry):
        j0 = 2 * k2
        j1 = j0 + 1
        unpack(j1, sa_b, da_b)
        pltpu.async_copy(h_hbm.at[sa_b], rows_b, sem_b)
        pltpu.make_async_copy(z_hbm.at[pl.ds(0, CH)], rows_a, sem_a).wait()
        pltpu.sync_copy(rows_a, acc.at[da_a], add=True)

        @pl.when(j0 + 2 < NCH)
        def _():
            unpack(j0 + 2, sa_a, da_a)
            pltpu.async_copy(h_hbm.at[sa_a], rows_a, sem_a)

        pltpu.make_async_copy(z_hbm.at[pl.ds(0, CH)], rows_b, sem_b).wait()
        pltpu.sync_copy(rows_b, acc.at[da_b], add=True)
        return carry

    lax.fori_loop(0, NCH // 2, body, 0)
    plsc.subcore_barrier()
    for t in range(RPS // CH):
        pltpu.sync_copy(acc.at[pl.ds(s * RPS + t * CH, CH)], rows_a)
        pltpu.sync_copy(rows_a, out_hbm.at[c, pl.ds(s * RPS + t * CH, CH)])


_TR = 1000


def _prep_body(x_ref, d_ref, nin_ref, nout_ref, h_ref):
    dd = d_ref[...]
    deg_out = dd[0, :, 0:1]
    deg_in = dd[1, :, 0:1]
    no = jnp.where(deg_out > 0, lax.rsqrt(jnp.maximum(deg_out, 1e-12)), 0.0)
    ni = jnp.where(deg_in > 0, lax.rsqrt(jnp.maximum(deg_in, 1e-12)), 0.0)
    nout_ref[...] = no
    nin_ref[...] = ni
    h_ref[...] = x_ref[...] * no


def _prep_call(x, degp):
    return pl.pallas_call(
        _prep_body,
        grid=(N // _TR,),
        in_specs=[
            pl.BlockSpec((_TR, D), lambda i: (i, 0)),
            pl.BlockSpec((NC, _TR, D), lambda i: (0, i, 0)),
        ],
        out_specs=[
            pl.BlockSpec((_TR, 1), lambda i: (i, 0)),
            pl.BlockSpec((_TR, 1), lambda i: (i, 0)),
            pl.BlockSpec((_TR, D), lambda i: (i, 0)),
        ],
        out_shape=[
            jax.ShapeDtypeStruct((N, 1), jnp.float32),
            jax.ShapeDtypeStruct((N, 1), jnp.float32),
            jax.ShapeDtypeStruct((NP, D), jnp.float32),
        ],
    )(x, degp)


def _layer_body(relu, scale_out, p_ref, nin_ref, nout_ref, w_ref, b_ref, o_ref):
    x = (p_ref[0] + p_ref[1]) * nin_ref[...]
    y = jnp.dot(x, w_ref[...], preferred_element_type=jnp.float32) + b_ref[...]
    if relu:
        y = jnp.maximum(y, 0.0)
    if scale_out:
        y = y * nout_ref[...]
    o_ref[...] = y


def _layer_call(p, nin, nout, W, b, relu, scale_out, out_rows):
    return pl.pallas_call(
        functools.partial(_layer_body, relu, scale_out),
        grid=(N // _TR,),
        in_specs=[
            pl.BlockSpec((NC, _TR, D), lambda i: (0, i, 0)),
            pl.BlockSpec((_TR, 1), lambda i: (i, 0)),
            pl.BlockSpec((_TR, 1), lambda i: (i, 0)),
            pl.BlockSpec((D, D), lambda i: (0, 0)),
            pl.BlockSpec((1, D), lambda i: (0, 0)),
        ],
        out_specs=pl.BlockSpec((_TR, D), lambda i: (i, 0)),
        out_shape=jax.ShapeDtypeStruct((out_rows, D), jnp.float32),
    )(p, nin, nout, W, b)


def kernel(x, edge_index, W1, b1, W2, b2, W3, b3):
    pad = EP - E
    pad_idx = (N + jnp.arange(pad, dtype=jnp.int32) % (NP - N))
    src_flat = jnp.concatenate([edge_index[0].astype(jnp.int32), pad_idx])
    dst_flat = jnp.concatenate([edge_index[1].astype(jnp.int32), pad_idx])
    packed = (src_flat | (dst_flat << 14)).reshape(NW, NCH, CH)
    src_d = src_flat.reshape(NS, NCHD, CHD)
    dst_d = dst_flat.reshape(NS, NCHD, CHD)
    z128 = jnp.zeros((NP, D), jnp.float32)
    ones128 = jnp.ones((CHD, D), jnp.float32)
    b1r = b1.reshape(1, D)
    b2r = b2.reshape(1, D)
    b3r = b3.reshape(1, D)

    degp = _deg_kernel(src_d, dst_d, z128, ones128)
    nin, nout, h = _prep_call(x, degp)

    p = _gs_kernel(h, packed, z128)
    h = _layer_call(p, nin, nout, W1, b1r, relu=True, scale_out=True, out_rows=NP)
    p = _gs_kernel(h, packed, z128)
    h = _layer_call(p, nin, nout, W2, b2r, relu=True, scale_out=True, out_rows=NP)
    p = _gs_kernel(h, packed, z128)
    out = _layer_call(p, nin, nout, W3, b3r, relu=False, scale_out=False, out_rows=N)
    return out


# speedup vs baseline: 9.4259x; 1.1214x over previous
import functools

import jax
import jax.numpy as jnp
from jax import lax
from jax.experimental import pallas as pl
from jax.experimental.pallas import tpu as pltpu
from jax.experimental.pallas import tpu_sc as plsc

N = 10000
E = 320000
D = 128
NC = 2
NS = 16
NW = NC * NS
CH = 128
NCH = 80
EP = NW * NCH * CH
CHD = 128
NCHD = EP // (NS * CHD)
NP = 10240
RPS = NP // NS

_mesh = plsc.VectorSubcoreMesh(core_axis_name="c", subcore_axis_name="s")


@functools.partial(
    pl.kernel,
    out_type=jax.ShapeDtypeStruct((NC, NP, D), jnp.float32),
    mesh=_mesh,
    scratch_types=[
        pltpu.VMEM((NCHD, CHD), jnp.int32),
        pltpu.VMEM((CHD, D), jnp.float32),
        pltpu.VMEM_SHARED((NP, D), jnp.float32),
    ],
)
def _deg_kernel(src_hbm, dst_hbm, z_hbm, ones_hbm, out_hbm,
                idx_v, buf_v, acc):
    c = lax.axis_index("c")
    s = lax.axis_index("s")

    @pl.when(c == 0)
    def _():
        pltpu.sync_copy(src_hbm.at[s], idx_v)

    @pl.when(c != 0)
    def _():
        pltpu.sync_copy(dst_hbm.at[s], idx_v)

    pltpu.sync_copy(z_hbm.at[pl.ds(0, CHD)], buf_v)
    for t in range(RPS // CHD):
        pltpu.sync_copy(buf_v, acc.at[pl.ds(s * RPS + t * CHD, CHD)])
    plsc.subcore_barrier()
    pltpu.sync_copy(ones_hbm, buf_v)

    def chunk(j, carry):
        pltpu.sync_copy(buf_v, acc.at[idx_v.at[j]], add=True)
        return carry

    lax.fori_loop(0, NCHD, chunk, 0)
    plsc.subcore_barrier()
    for t in range(RPS // CHD):
        pltpu.sync_copy(acc.at[pl.ds(s * RPS + t * CHD, CHD)], buf_v)
        pltpu.sync_copy(buf_v, out_hbm.at[c, pl.ds(s * RPS + t * CHD, CHD)])


@functools.partial(
    pl.kernel,
    out_type=jax.ShapeDtypeStruct((NC, NP, D), jnp.float32),
    mesh=_mesh,
    scratch_types=[
        pltpu.VMEM((NCH, CH), jnp.int32),
        pltpu.VMEM((CH,), jnp.int32),
        pltpu.VMEM((CH,), jnp.int32),
        pltpu.VMEM((CH,), jnp.int32),
        pltpu.VMEM((CH,), jnp.int32),
        pltpu.VMEM((CH, D), jnp.float32),
        pltpu.VMEM((CH, D), jnp.float32),
        pltpu.SemaphoreType.DMA,
        pltpu.SemaphoreType.DMA,
        pltpu.VMEM_SHARED((NP, D), jnp.float32),
    ],
)
def _gs_kernel(h_hbm, pk_hbm, z_hbm, out_hbm,
               pk_v, sa_a, da_a, sa_b, da_b, rows_a, rows_b,
               sem_a, sem_b, acc):
    c = lax.axis_index("c")
    s = lax.axis_index("s")
    w = s * NC + c
    pltpu.sync_copy(pk_hbm.at[w], pk_v)
    pltpu.sync_copy(z_hbm.at[pl.ds(0, CH)], rows_a)
    for t in range(RPS // CH):
        pltpu.sync_copy(rows_a, acc.at[pl.ds(s * RPS + t * CH, CH)])
    plsc.subcore_barrier()

    def unpack(j, sa, da):
        for q in range(CH // 16):
            v = pk_v[j, pl.ds(q * 16, 16)]
            sa[pl.ds(q * 16, 16)] = lax.bitwise_and(v, 0x3FFF)
            da[pl.ds(q * 16, 16)] = lax.shift_right_logical(v, 14)

    unpack(0, sa_a, da_a)
    pltpu.async_copy(h_hbm.at[sa_a], rows_a, sem_a)

    def body(k2, carry):
        j0 = 2 * k2
        j1 = j0 + 1
        unpack(j1, sa_b, da_b)
        pltpu.async_copy(h_hbm.at[sa_b], rows_b, sem_b)
        pltpu.make_async_copy(z_hbm.at[pl.ds(0, CH)], rows_a, sem_a).wait()
        pltpu.sync_copy(rows_a, acc.at[da_a], add=True)

        @pl.when(j0 + 2 < NCH)
        def _():
            unpack(j0 + 2, sa_a, da_a)
            pltpu.async_copy(h_hbm.at[sa_a], rows_a, sem_a)

        pltpu.make_async_copy(z_hbm.at[pl.ds(0, CH)], rows_b, sem_b).wait()
        pltpu.sync_copy(rows_b, acc.at[da_b], add=True)
        return carry

    lax.fori_loop(0, NCH // 2, body, 0)
    plsc.subcore_barrier()
    for t in range(RPS // CH):
        pltpu.sync_copy(acc.at[pl.ds(s * RPS + t * CH, CH)], rows_a)
        pltpu.sync_copy(rows_a, out_hbm.at[c, pl.ds(s * RPS + t * CH, CH)])


_TR = 1000


def _prep_body(x_ref, d_ref, nin_ref, nout_ref, h_ref):
    dd = d_ref[...]
    deg_out = dd[0, :, 0:1]
    deg_in = dd[1, :, 0:1]
    no = jnp.where(deg_out > 0, lax.rsqrt(jnp.maximum(deg_out, 1e-12)), 0.0)
    ni = jnp.where(deg_in > 0, lax.rsqrt(jnp.maximum(deg_in, 1e-12)), 0.0)
    nout_ref[...] = no
    nin_ref[...] = ni
    h_ref[...] = x_ref[...] * no


def _prep_call(x, degp):
    return pl.pallas_call(
        _prep_body,
        grid=(N // _TR,),
        in_specs=[
            pl.BlockSpec((_TR, D), lambda i: (i, 0)),
            pl.BlockSpec((NC, _TR, D), lambda i: (0, i, 0)),
        ],
        out_specs=[
            pl.BlockSpec((_TR, 1), lambda i: (i, 0)),
            pl.BlockSpec((_TR, 1), lambda i: (i, 0)),
            pl.BlockSpec((_TR, D), lambda i: (i, 0)),
        ],
        out_shape=[
            jax.ShapeDtypeStruct((N, 1), jnp.float32),
            jax.ShapeDtypeStruct((N, 1), jnp.float32),
            jax.ShapeDtypeStruct((NP, D), jnp.float32),
        ],
    )(x, degp)


def _layer_body(relu, scale_out, p_ref, nin_ref, nout_ref, w_ref, b_ref, o_ref):
    x = (p_ref[0] + p_ref[1]) * nin_ref[...]
    y = jnp.dot(x, w_ref[...], preferred_element_type=jnp.float32) + b_ref[...]
    if relu:
        y = jnp.maximum(y, 0.0)
    if scale_out:
        y = y * nout_ref[...]
    o_ref[...] = y


def _layer_call(p, nin, nout, W, b, relu, scale_out, out_rows):
    return pl.pallas_call(
        functools.partial(_layer_body, relu, scale_out),
        grid=(N // _TR,),
        in_specs=[
            pl.BlockSpec((NC, _TR, D), lambda i: (0, i, 0)),
            pl.BlockSpec((_TR, 1), lambda i: (i, 0)),
            pl.BlockSpec((_TR, 1), lambda i: (i, 0)),
            pl.BlockSpec((D, D), lambda i: (0, 0)),
            pl.BlockSpec((1, D), lambda i: (0, 0)),
        ],
        out_specs=pl.BlockSpec((_TR, D), lambda i: (i, 0)),
        out_shape=jax.ShapeDtypeStruct((out_rows, D), jnp.float32),
    )(p, nin, nout, W, b)


def kernel(x, edge_index, W1, b1, W2, b2, W3, b3):
    pad = EP - E
    pad_idx = (N + jnp.arange(pad, dtype=jnp.int32) % (NP - N))
    src_flat = jnp.concatenate([edge_index[0].astype(jnp.int32), pad_idx])
    dst_flat = jnp.concatenate([edge_index[1].astype(jnp.int32), pad_idx])
    packed = (src_flat | (dst_flat << 14)).reshape(NW, NCH, CH)
    src_d = src_flat.reshape(NS, NCHD, CHD)
    dst_d = dst_flat.reshape(NS, NCHD, CHD)
    z128 = jnp.zeros((NP, D), jnp.float32)
    ones128 = jnp.ones((CHD, D), jnp.float32)
    b1r = b1.reshape(1, D)
    b2r = b2.reshape(1, D)
    b3r = b3.reshape(1, D)

    degp = _deg_kernel(src_d, dst_d, z128, ones128)
    nin, nout, h = _prep_call(x, degp)

    p = _gs_kernel(h, packed, z128)
    h = _layer_call(p, nin, nout, W1, b1r, relu=True, scale_out=True, out_rows=NP)
    p = _gs_kernel(h, packed, z128)
    h = _layer_call(p, nin, nout, W2, b2r, relu=True, scale_out=True, out_rows=NP)
    p = _gs_kernel(h, packed, z128)
    out = _layer_call(p, nin, nout, W3, b3r, relu=False, scale_out=False, out_rows=N)
    return out
